# Initial kernel scaffold; baseline (speedup 1.0000x reference)
import jax
import jax.numpy as jnp
from jax.experimental import pallas as pl


def kernel(x, edge_index, edge_type, Wl1, bl1, Wr1, br1, Wp1, bp1, Wt1, bt1, Wl2, bl2, Wr2, br2, Wp2, bp2, Wt2, bt2):
    raise NotImplementedError("write your pallas kernel here")


# speedup vs baseline: 1.4004x; 1.4004x over previous
import functools
import jax
import jax.numpy as jnp
from jax import lax
from jax.experimental import pallas as pl
from jax.experimental.pallas import tpu as pltpu
from jax.experimental.pallas import tpu_sc as plsc

N = 10000
E = 320000
D = 128
R = 4
EPS = 1e-10

NC = 2
NS = 16
SEG = N * R
SEG_HALF = SEG // NC
ACC_ROWS = SEG_HALF + 96
STRIPE = ACC_ROWS // NS
NFL = 10
FSTRIPE = SEG_HALF // NFL
B = 80
EPW = E // NS
NBATCH = EPW // B
CH = D // 2


def _seg_body(with_counts, hv, srcr, dstr, etr, zrows, znc, onesh, *rest):
    if with_counts:
        (s_out0, s_out1, cnt_out, acc, cnta,
         srcb, dstb, etb, gidxb, sidxb, rowsb, onesb, sem) = rest
    else:
        (s_out0, s_out1, acc,
         srcb, dstb, etb, gidxb, sidxb, rowsb, sem) = rest
        cnt_out = cnta = onesb = None
    s_outs = (s_out0, s_out1)

    c = lax.axis_index("c")
    s = lax.axis_index("s")
    base = c * SEG_HALF

    if with_counts:
        pltpu.sync_copy(onesh, onesb)

    for p in range(2):
        pltpu.sync_copy(zrows, acc.at[pl.ds(s * STRIPE, STRIPE), :])
        if with_counts and p == 0:
            pltpu.sync_copy(znc, cnta.at[pl.ds(s * STRIPE, STRIPE), :])
        plsc.subcore_barrier()

        def batch(i, _):
            e0 = s * EPW + i * B
            pltpu.sync_copy(srcr.at[pl.ds(e0, B)], srcb)
            pltpu.sync_copy(dstr.at[pl.ds(e0, B)], dstb)
            pltpu.sync_copy(etr.at[pl.ds(e0, B)], etb)

            def lane(j, _):
                dsj = pl.ds(j * 16, 16)
                sv = srcb[dsj]
                dv = dstb[dsj]
                ev = etb[dsj]
                si = dv * R + ev - base
                ok = (si >= 0) & (si < SEG_HALF)
                sidxb[dsj] = jnp.where(ok, si, SEG_HALF)
                gidxb[dsj] = sv * 2 + p
                return 0

            lax.fori_loop(0, B // 16, lane, 0)
            pltpu.async_copy(hv.at[gidxb], rowsb, sem).wait()
            pltpu.sync_copy(rowsb, acc.at[sidxb], add=True)
            if with_counts and p == 0:
                pltpu.sync_copy(onesb, cnta.at[sidxb], add=True)
            return 0

        lax.fori_loop(0, NBATCH, batch, 0)
        plsc.subcore_barrier()

        @pl.when(s < NFL)
        def _flush():
            r0 = s * FSTRIPE
            pltpu.sync_copy(
                acc.at[pl.ds(r0, FSTRIPE), :],
                s_outs[p].at[pl.ds(base + r0, FSTRIPE), :])
            if with_counts and p == 0:
                pltpu.sync_copy(cnta.at[pl.ds(r0, FSTRIPE), :],
                                cnt_out.at[pl.ds(base + r0, FSTRIPE), :])

        plsc.subcore_barrier()


@functools.lru_cache(maxsize=None)
def _make_seg(with_counts):
    mesh = plsc.VectorSubcoreMesh(core_axis_name="c", subcore_axis_name="s",
                                  num_cores=NC, num_subcores=NS)
    out_type = [jax.ShapeDtypeStruct((SEG, CH), jnp.float32),
                jax.ShapeDtypeStruct((SEG, CH), jnp.float32)]
    scratch = [
        pltpu.VMEM_SHARED((ACC_ROWS, CH), jnp.float32),
    ]
    if with_counts:
        out_type.append(jax.ShapeDtypeStruct((SEG, 16), jnp.float32))
        scratch.append(pltpu.VMEM_SHARED((ACC_ROWS, 16), jnp.float32))
    scratch += [
        pltpu.VMEM((B,), jnp.int32),
        pltpu.VMEM((B,), jnp.int32),
        pltpu.VMEM((B,), jnp.int32),
        pltpu.VMEM((B,), jnp.int32),
        pltpu.VMEM((B,), jnp.int32),
        pltpu.VMEM((B, CH), jnp.float32),
    ]
    if with_counts:
        scratch.append(pltpu.VMEM((B, 16), jnp.float32))
    scratch.append(pltpu.SemaphoreType.DMA)
    return pl.kernel(
        functools.partial(_seg_body, with_counts),
        out_type=tuple(out_type),
        mesh=mesh,
        scratch_types=tuple(scratch),
        compiler_params=pltpu.CompilerParams(use_tc_tiling_on_sc=False),
    )


def _dense_body(xin, prev, s, cnt, wl, wr, wpa, wpb, wta, wtb,
                b1, bp, bt, h_out, g_out):
    inv = 1.0 / (cnt[...] + EPS)
    acc = jnp.dot(xin[...], wl[...], preferred_element_type=jnp.float32)
    for r in range(R):
        upd = s[:, r * D:(r + 1) * D] * inv[:, r:r + 1]
        acc = acc + jnp.dot(upd, wr[r * D:(r + 1) * D, :],
                            preferred_element_type=jnp.float32)
    h = jax.nn.sigmoid(acc + b1[...])
    pv = prev[...]
    pa = jax.nn.relu(
        jnp.dot(h, wpa[...], preferred_element_type=jnp.float32)
        + jnp.dot(pv, wpb[...], preferred_element_type=jnp.float32)
        + bp[...])
    ga = jax.nn.sigmoid(
        jnp.dot(h, wta[...], preferred_element_type=jnp.float32)
        + jnp.dot(pv, wtb[...], preferred_element_type=jnp.float32)
        + bt[...])
    h_out[...] = h
    g_out[...] = ga * pa + (1.0 - ga) * h


_BN = 1000


def _dense_call(xin, prev, s, cnt4, wlT, wrT, wpaT, wpbT, wtaT, wtbT,
                b1, bp, bt):
    grid = (N // _BN,)
    row = lambda i: (i, 0)
    const = lambda i: (0, 0)
    return pl.pallas_call(
        _dense_body,
        grid=grid,
        in_specs=[
            pl.BlockSpec((_BN, D), row),
            pl.BlockSpec((_BN, D), row),
            pl.BlockSpec((_BN, R * D), row),
            pl.BlockSpec((_BN, R), row),
            pl.BlockSpec((D, D), const),
            pl.BlockSpec((R * D, D), const),
            pl.BlockSpec((D, D), const),
            pl.BlockSpec((D, D), const),
            pl.BlockSpec((D, D), const),
            pl.BlockSpec((D, D), const),
            pl.BlockSpec((1, D), const),
            pl.BlockSpec((1, D), const),
            pl.BlockSpec((1, D), const),
        ],
        out_specs=[
            pl.BlockSpec((_BN, D), row),
            pl.BlockSpec((_BN, D), row),
        ],
        out_shape=[
            jax.ShapeDtypeStruct((N, D), jnp.float32),
            jax.ShapeDtypeStruct((N, D), jnp.float32),
        ],
    )(xin, prev, s, cnt4, wlT, wrT, wpaT, wpbT, wtaT, wtbT, b1, bp, bt)


def kernel(x, edge_index, edge_type,
           Wl1, bl1, Wr1, br1, Wp1, bp1, Wt1, bt1,
           Wl2, bl2, Wr2, br2, Wp2, bp2, Wt2, bt2):
    src = edge_index[0]
    dst = edge_index[1]

    zrows = jnp.zeros((STRIPE, CH), jnp.float32)
    znc = jnp.zeros((STRIPE, 16), jnp.float32)
    onesh = jnp.zeros((B, 16), jnp.float32).at[:, 0].set(1.0)

    s1a, s1b, cnt = _make_seg(True)(x.reshape(2 * N, CH), src, dst, edge_type,
                                    zrows, znc, onesh)
    s1 = jnp.concatenate([s1a, s1b], axis=-1)
    cnt4 = cnt[:, 0].reshape(N, R)
    h1, g1 = _dense_call(
        x, x, s1.reshape(N, R * D), cnt4,
        Wl1.T, Wr1.T, Wp1[:, :D].T, Wp1[:, D:].T, Wt1[:, :D].T, Wt1[:, D:].T,
        (bl1 + br1).reshape(1, D), bp1.reshape(1, D), bt1.reshape(1, D))

    s2a, s2b = _make_seg(False)(g1.reshape(2 * N, CH), src, dst, edge_type,
                                zrows, znc, onesh)
    s2 = jnp.concatenate([s2a, s2b], axis=-1)
    _, g2 = _dense_call(
        g1, h1, s2.reshape(N, R * D), cnt4,
        Wl2.T, Wr2.T, Wp2[:, :D].T, Wp2[:, D:].T, Wt2[:, :D].T, Wt2[:, D:].T,
        (bl2 + br2).reshape(1, D), bp2.reshape(1, D), bt2.reshape(1, D))
    return g2


# speedup vs baseline: 3.0103x; 2.1496x over previous
import functools
import jax
import jax.numpy as jnp
from jax import lax
from jax.experimental import pallas as pl
from jax.experimental.pallas import tpu as pltpu
from jax.experimental.pallas import tpu_sc as plsc

N = 10000
E = 320000
D = 128
R = 4
EPS = 1e-10

NC = 2
NS = 16
SEG = N * R
SEG_HALF = SEG // NC
ACC_ROWS = SEG_HALF + 96
STRIPE = ACC_ROWS // NS
NFL = 10
FSTRIPE = SEG_HALF // NFL
B = 80
EPW = E // NS
NBATCH = EPW // B
CH = D // 2


def _seg_body(with_counts, hv, srcr, dstr, etr, zrows, znc, onesh, *rest):
    if with_counts:
        (s_out0, s_out1, cnt_out, acc, cnta,
         src0, dst0, et0, gid0, sid0, rows0,
         src1, dst1, et1, gid1, sid1, rows1,
         onesb, semi0, semi1, semg0, semg1) = rest
    else:
        (s_out0, s_out1, acc,
         src0, dst0, et0, gid0, sid0, rows0,
         src1, dst1, et1, gid1, sid1, rows1,
         semi0, semi1, semg0, semg1) = rest
        cnt_out = cnta = onesb = None
    s_outs = (s_out0, s_out1)
    slots = ((src0, dst0, et0, gid0, sid0, rows0, semi0, semg0),
             (src1, dst1, et1, gid1, sid1, rows1, semi1, semg1))

    c = lax.axis_index("c")
    s = lax.axis_index("s")
    base = c * SEG_HALF

    if with_counts:
        pltpu.sync_copy(onesh, onesb)

    def issue_idx(slot, i):
        srcb, dstb, etb = slot[0], slot[1], slot[2]
        semi = slot[6]
        e0 = jnp.minimum(s * EPW + i * B, E - B)
        pltpu.async_copy(srcr.at[pl.ds(e0, B)], srcb, semi)
        pltpu.async_copy(dstr.at[pl.ds(e0, B)], dstb, semi)
        pltpu.async_copy(etr.at[pl.ds(e0, B)], etb, semi)

    def wait_idx(slot):
        srcb, dstb, etb = slot[0], slot[1], slot[2]
        semi = slot[6]
        pltpu.make_async_copy(srcr.at[pl.ds(0, B)], srcb, semi).wait()
        pltpu.make_async_copy(srcr.at[pl.ds(0, B)], dstb, semi).wait()
        pltpu.make_async_copy(srcr.at[pl.ds(0, B)], etb, semi).wait()

    def compute(slot, p):
        srcb, dstb, etb, gidb, sidb = slot[:5]

        def lane(j, _):
            dsj = pl.ds(j * 16, 16)
            sv = srcb[dsj]
            dv = dstb[dsj]
            ev = etb[dsj]
            si = dv * R + ev - base
            ok = (si >= 0) & (si < SEG_HALF)
            sidb[dsj] = jnp.where(ok, si, SEG_HALF)
            gidb[dsj] = sv * 2 + p
            return 0

        lax.fori_loop(0, B // 16, lane, 0)

    def issue_gather(slot):
        pltpu.async_copy(hv.at[slot[3]], slot[5], slot[7])

    def finish_scatter(slot, p):
        sidb, rowsb, semg = slot[4], slot[5], slot[7]
        pltpu.make_async_copy(hv.at[pl.ds(0, B)], rowsb, semg).wait()
        pltpu.sync_copy(rowsb, acc.at[sidb], add=True)
        if with_counts and p == 0:
            pltpu.sync_copy(onesb, cnta.at[sidb], add=True)

    for p in range(2):
        pltpu.sync_copy(zrows, acc.at[pl.ds(s * STRIPE, STRIPE), :])
        if with_counts and p == 0:
            pltpu.sync_copy(znc, cnta.at[pl.ds(s * STRIPE, STRIPE), :])
        plsc.subcore_barrier()

        issue_idx(slots[0], 0)
        issue_idx(slots[1], 1)

        def step(jb, _):
            for b in range(2):
                i = 2 * jb + b
                slot = slots[b]
                wait_idx(slot)
                compute(slot, p)
                issue_gather(slot)
                issue_idx(slot, i + 2)

                @pl.when(i > 0)
                def _():
                    finish_scatter(slots[1 - b], p)

            return 0

        lax.fori_loop(0, NBATCH // 2, step, 0)
        finish_scatter(slots[1], p)
        wait_idx(slots[0])
        wait_idx(slots[1])
        plsc.subcore_barrier()

        @pl.when(s < NFL)
        def _flush():
            r0 = s * FSTRIPE
            pltpu.sync_copy(
                acc.at[pl.ds(r0, FSTRIPE), :],
                s_outs[p].at[pl.ds(base + r0, FSTRIPE), :])
            if with_counts and p == 0:
                pltpu.sync_copy(cnta.at[pl.ds(r0, FSTRIPE), :],
                                cnt_out.at[pl.ds(base + r0, FSTRIPE), :])

        plsc.subcore_barrier()


@functools.lru_cache(maxsize=None)
def _make_seg(with_counts):
    mesh = plsc.VectorSubcoreMesh(core_axis_name="c", subcore_axis_name="s",
                                  num_cores=NC, num_subcores=NS)
    out_type = [jax.ShapeDtypeStruct((SEG, CH), jnp.float32),
                jax.ShapeDtypeStruct((SEG, CH), jnp.float32)]
    scratch = [
        pltpu.VMEM_SHARED((ACC_ROWS, CH), jnp.float32),
    ]
    if with_counts:
        out_type.append(jax.ShapeDtypeStruct((SEG, 16), jnp.float32))
        scratch.append(pltpu.VMEM_SHARED((ACC_ROWS, 16), jnp.float32))
    for _slot in range(2):
        scratch += [
            pltpu.VMEM((B,), jnp.int32),
            pltpu.VMEM((B,), jnp.int32),
            pltpu.VMEM((B,), jnp.int32),
            pltpu.VMEM((B,), jnp.int32),
            pltpu.VMEM((B,), jnp.int32),
            pltpu.VMEM((B, CH), jnp.float32),
        ]
    if with_counts:
        scratch.append(pltpu.VMEM((B, 16), jnp.float32))
    scratch += [pltpu.SemaphoreType.DMA] * 4
    return pl.kernel(
        functools.partial(_seg_body, with_counts),
        out_type=tuple(out_type),
        mesh=mesh,
        scratch_types=tuple(scratch),
        compiler_params=pltpu.CompilerParams(use_tc_tiling_on_sc=False),
    )


def _dense_body(xin, prev, s, cnt, wl, wr, wpa, wpb, wta, wtb,
                b1, bp, bt, h_out, g_out):
    inv = 1.0 / (cnt[...] + EPS)
    acc = jnp.dot(xin[...], wl[...], preferred_element_type=jnp.float32)
    for r in range(R):
        upd = s[:, r * D:(r + 1) * D] * inv[:, r:r + 1]
        acc = acc + jnp.dot(upd, wr[r * D:(r + 1) * D, :],
                            preferred_element_type=jnp.float32)
    h = jax.nn.sigmoid(acc + b1[...])
    pv = prev[...]
    pa = jax.nn.relu(
        jnp.dot(h, wpa[...], preferred_element_type=jnp.float32)
        + jnp.dot(pv, wpb[...], preferred_element_type=jnp.float32)
        + bp[...])
    ga = jax.nn.sigmoid(
        jnp.dot(h, wta[...], preferred_element_type=jnp.float32)
        + jnp.dot(pv, wtb[...], preferred_element_type=jnp.float32)
        + bt[...])
    h_out[...] = h
    g_out[...] = ga * pa + (1.0 - ga) * h


_BN = 1000


def _dense_call(xin, prev, s, cnt4, wlT, wrT, wpaT, wpbT, wtaT, wtbT,
                b1, bp, bt):
    grid = (N // _BN,)
    row = lambda i: (i, 0)
    const = lambda i: (0, 0)
    return pl.pallas_call(
        _dense_body,
        grid=grid,
        in_specs=[
            pl.BlockSpec((_BN, D), row),
            pl.BlockSpec((_BN, D), row),
            pl.BlockSpec((_BN, R * D), row),
            pl.BlockSpec((_BN, R), row),
            pl.BlockSpec((D, D), const),
            pl.BlockSpec((R * D, D), const),
            pl.BlockSpec((D, D), const),
            pl.BlockSpec((D, D), const),
            pl.BlockSpec((D, D), const),
            pl.BlockSpec((D, D), const),
            pl.BlockSpec((1, D), const),
            pl.BlockSpec((1, D), const),
            pl.BlockSpec((1, D), const),
        ],
        out_specs=[
            pl.BlockSpec((_BN, D), row),
            pl.BlockSpec((_BN, D), row),
        ],
        out_shape=[
            jax.ShapeDtypeStruct((N, D), jnp.float32),
            jax.ShapeDtypeStruct((N, D), jnp.float32),
        ],
    )(xin, prev, s, cnt4, wlT, wrT, wpaT, wpbT, wtaT, wtbT, b1, bp, bt)


def kernel(x, edge_index, edge_type,
           Wl1, bl1, Wr1, br1, Wp1, bp1, Wt1, bt1,
           Wl2, bl2, Wr2, br2, Wp2, bp2, Wt2, bt2):
    src = edge_index[0]
    dst = edge_index[1]

    zrows = jnp.zeros((STRIPE, CH), jnp.float32)
    znc = jnp.zeros((STRIPE, 16), jnp.float32)
    onesh = jnp.zeros((B, 16), jnp.float32).at[:, 0].set(1.0)

    s1a, s1b, cnt = _make_seg(True)(x.reshape(2 * N, CH), src, dst, edge_type,
                                    zrows, znc, onesh)
    s1 = jnp.concatenate([s1a, s1b], axis=-1)
    cnt4 = cnt[:, 0].reshape(N, R)
    h1, g1 = _dense_call(
        x, x, s1.reshape(N, R * D), cnt4,
        Wl1.T, Wr1.T, Wp1[:, :D].T, Wp1[:, D:].T, Wt1[:, :D].T, Wt1[:, D:].T,
        (bl1 + br1).reshape(1, D), bp1.reshape(1, D), bt1.reshape(1, D))

    s2a, s2b = _make_seg(False)(g1.reshape(2 * N, CH), src, dst, edge_type,
                                zrows, znc, onesh)
    s2 = jnp.concatenate([s2a, s2b], axis=-1)
    _, g2 = _dense_call(
        g1, h1, s2.reshape(N, R * D), cnt4,
        Wl2.T, Wr2.T, Wp2[:, :D].T, Wp2[:, D:].T, Wt2[:, :D].T, Wt2[:, D:].T,
        (bl2 + br2).reshape(1, D), bp2.reshape(1, D), bt2.reshape(1, D))
    return g2
